# Initial kernel scaffold; baseline (speedup 1.0000x reference)
import jax
import jax.numpy as jnp
from jax.experimental import pallas as pl


def kernel(x, edge_index, W1, att_src1, att_dst1, b1, W2, att_src2, att_dst2, b2):
    raise NotImplementedError("write your pallas kernel here")


# speedup vs baseline: 3.9150x; 3.9150x over previous
import functools

import jax
import jax.numpy as jnp
from jax import lax
from jax.experimental import pallas as pl
from jax.experimental.pallas import tpu as pltpu

N = 10000
E = 320000
NFEAT = 128
NHID = 16
HEADS = 8
NCLASS = 40

N_PAD = 10016
E2 = E + N
E_PAD = 331776


def _proj_kernel(x_ref, w_ref, asrc_ref, adst_ref, g_ref, hp_ref, as_ref, ad_ref, mx_ref):
    hp = jnp.dot(x_ref[...], w_ref[...], preferred_element_type=jnp.float32)
    hp_ref[...] = hp
    g = g_ref[...]
    a_s = jnp.dot(hp * asrc_ref[...], g, preferred_element_type=jnp.float32)
    a_d = jnp.dot(hp * adst_ref[...], g, preferred_element_type=jnp.float32)
    as_ref[...] = a_s
    ad_ref[...] = a_d
    rows = lax.broadcasted_iota(jnp.int32, (N_PAD, 1), 0)
    valid = rows < N
    neg = jnp.float32(-3e38)
    mx_s = jnp.max(jnp.where(valid, a_s, neg), axis=0, keepdims=True)
    mx_d = jnp.max(jnp.where(valid, a_d, neg), axis=0, keepdims=True)
    mx_ref[...] = jnp.concatenate([mx_s, mx_d], axis=1)


def _project(x_pad, w, a_src, a_dst, heads, dim):
    hd = heads * dim
    g = (lax.broadcasted_iota(jnp.int32, (hd, heads), 0) // dim ==
         lax.broadcasted_iota(jnp.int32, (hd, heads), 1)).astype(jnp.float32)
    return pl.pallas_call(
        _proj_kernel,
        out_shape=(
            jax.ShapeDtypeStruct((N_PAD, hd), jnp.float32),
            jax.ShapeDtypeStruct((N_PAD, heads), jnp.float32),
            jax.ShapeDtypeStruct((N_PAD, heads), jnp.float32),
            jax.ShapeDtypeStruct((1, 2 * heads), jnp.float32),
        ),
    )(x_pad, w, a_src, a_dst, g)


def kernel(x, edge_index, W1, att_src1, att_dst1, b1, W2, att_src2, att_dst2, b2):
    x_pad = jnp.zeros((N_PAD, NFEAT), jnp.float32).at[:N].set(x)
    loop = jnp.arange(N, dtype=jnp.int32)
    src = jnp.concatenate([edge_index[0], loop])
    dst = jnp.concatenate([edge_index[1], loop])

    hp1, as1, ad1, mx1 = _project(
        x_pad, W1, att_src1.reshape(1, -1), att_dst1.reshape(1, -1), HEADS, NHID)

    alpha = as1[src] + ad1[dst]
    alpha = jnp.where(alpha >= 0, alpha, 0.2 * alpha)
    s1 = mx1[0, :HEADS] + mx1[0, HEADS:]
    s1 = jnp.where(s1 >= 0, s1, 0.2 * s1)
    ex = jnp.exp(alpha - s1[None, :])
    denom = jax.ops.segment_sum(ex, dst, num_segments=N_PAD)
    coef = ex / (denom[dst] + 1e-16)
    msg = hp1[src].reshape(E2, HEADS, NHID) * coef[:, :, None]
    out1 = jax.ops.segment_sum(msg.reshape(E2, HEADS * NHID), dst, num_segments=N_PAD)
    h1 = out1 + b1[None, :]
    h1 = jnp.where(h1 > 0, h1, jnp.exp(jnp.minimum(h1, 0.0)) - 1.0)

    hp2, as2, ad2, mx2 = _project(
        h1, W2, att_src2.reshape(1, -1), att_dst2.reshape(1, -1), 1, NCLASS)

    alpha2 = as2[src] + ad2[dst]
    alpha2 = jnp.where(alpha2 >= 0, alpha2, 0.2 * alpha2)
    s2 = mx2[0, :1] + mx2[0, 1:]
    s2 = jnp.where(s2 >= 0, s2, 0.2 * s2)
    ex2 = jnp.exp(alpha2 - s2[None, :])
    denom2 = jax.ops.segment_sum(ex2, dst, num_segments=N_PAD)
    coef2 = ex2 / (denom2[dst] + 1e-16)
    msg2 = hp2[src] * coef2
    out2 = jax.ops.segment_sum(msg2, dst, num_segments=N_PAD)
    h2 = out2[:N] + b2[None, :]
    return jax.nn.log_softmax(h2, axis=1)
